# Initial kernel scaffold; baseline (speedup 1.0000x reference)
import functools

import jax
import jax.numpy as jnp
from jax import lax
from jax.experimental import pallas as pl
from jax.experimental.pallas import tpu as pltpu
from jax.experimental.pallas import tpu_sc as plsc


def _embedding_bag_mean(x, table):
    B, H = x.shape
    _, D = table.shape
    info = plsc.get_sparse_core_info()
    NC, NS, L = info.num_cores, info.num_subcores, info.num_lanes
    NW = NC * NS
    b_per_w = B // NW
    CHUNK = 2
    IPC = CHUNK * H
    n_chunks = b_per_w // CHUNK

    x_r = x.reshape(NW, n_chunks, IPC).astype(jnp.int32)
    mesh = plsc.VectorSubcoreMesh(core_axis_name="c", subcore_axis_name="s")

    @functools.partial(
        pl.kernel,
        mesh=mesh,
        out_type=jax.ShapeDtypeStruct((B, D), jnp.float32),
        scratch_types=[
            pltpu.VMEM((n_chunks, IPC), jnp.int32),
            pltpu.VMEM((IPC, D), jnp.float32),
            pltpu.VMEM((b_per_w, D), jnp.float32),
            pltpu.SemaphoreType.DMA,
        ],
    )
    def emb_kernel(x_hbm, table_hbm, out_hbm, idx_v, rows_v, h_v, sem):
        wid = lax.axis_index("s") * NC + lax.axis_index("c")
        pltpu.sync_copy(x_hbm.at[wid], idx_v)
        scale = jnp.float32(1.0 / H)

        def body(c, carry):
            pltpu.async_copy(table_hbm.at[idx_v.at[c]], rows_v, sem).wait()
            for j in range(CHUNK):
                for dd in range(D // L):
                    sl = pl.ds(dd * L, L)
                    acc = rows_v[j * H, sl]
                    for k in range(1, H):
                        acc = acc + rows_v[j * H + k, sl]
                    h_v[c * CHUNK + j, sl] = acc * scale
            return carry

        lax.fori_loop(0, n_chunks, body, 0)
        pltpu.sync_copy(h_v, out_hbm.at[pl.ds(wid * b_per_w, b_per_w)])

    return emb_kernel(x_r, table)


def _mlp_softmax(h, W1, b1, W2, b2):
    B, D = h.shape
    BLK = 2048

    def body(h_ref, w1_ref, b1_ref, w2_ref, b2_ref, o_ref):
        z = jnp.dot(h_ref[...], w1_ref[...], preferred_element_type=jnp.float32)
        z = z + b1_ref[...]
        z = jnp.dot(z, w2_ref[...], preferred_element_type=jnp.float32)
        z = z + b2_ref[...]
        z = z - jnp.max(z, axis=1, keepdims=True)
        e = jnp.exp(z)
        o_ref[...] = e / jnp.sum(e, axis=1, keepdims=True)

    return pl.pallas_call(
        body,
        grid=(B // BLK,),
        in_specs=[
            pl.BlockSpec((BLK, D), lambda i: (i, 0)),
            pl.BlockSpec((D, D), lambda i: (0, 0)),
            pl.BlockSpec((1, D), lambda i: (0, 0)),
            pl.BlockSpec((D, D), lambda i: (0, 0)),
            pl.BlockSpec((1, D), lambda i: (0, 0)),
        ],
        out_specs=pl.BlockSpec((BLK, D), lambda i: (i, 0)),
        out_shape=jax.ShapeDtypeStruct((B, D), jnp.float32),
    )(h, W1.T, b1.reshape(1, D), W2.T, b2.reshape(1, D))


def kernel(x, table, W1, b1, W2, b2):
    h = _embedding_bag_mean(x, table)
    return _mlp_softmax(h, W1, b1, W2, b2)


# speedup vs baseline: 2.0380x; 2.0380x over previous
import functools

import jax
import jax.numpy as jnp
from jax import lax
from jax.experimental import pallas as pl
from jax.experimental.pallas import tpu as pltpu
from jax.experimental.pallas import tpu_sc as plsc


def _embedding_bag_mean(x, table):
    B, H = x.shape
    _, D = table.shape
    info = plsc.get_sparse_core_info()
    NC, NS, L = info.num_cores, info.num_subcores, info.num_lanes
    NW = NC * NS
    b_per_w = B // NW
    CHUNK = 2
    IPC = CHUNK * H
    n_chunks = b_per_w // CHUNK

    x_r = x.reshape(NW, n_chunks, IPC).astype(jnp.int32)
    mesh = plsc.VectorSubcoreMesh(core_axis_name="c", subcore_axis_name="s")

    @functools.partial(
        pl.kernel,
        mesh=mesh,
        out_type=jax.ShapeDtypeStruct((B, D), jnp.float32),
        scratch_types=[
            pltpu.VMEM((n_chunks, IPC), jnp.int32),
            pltpu.VMEM((IPC, D), jnp.float32),
            pltpu.VMEM((b_per_w, D), jnp.float32),
            pltpu.SemaphoreType.DMA,
        ],
        compiler_params=pltpu.CompilerParams(use_tc_tiling_on_sc=False),
    )
    def emb_kernel(x_hbm, table_hbm, out_hbm, idx_v, rows_v, h_v, sem):
        wid = lax.axis_index("s") * NC + lax.axis_index("c")
        pltpu.sync_copy(x_hbm.at[wid], idx_v)
        scale = jnp.float32(1.0 / H)

        def body(c, carry):
            pltpu.async_copy(table_hbm.at[idx_v.at[c]], rows_v, sem).wait()
            for j in range(CHUNK):
                for dd in range(D // L):
                    sl = pl.ds(dd * L, L)
                    acc = rows_v[j * H, sl]
                    for k in range(1, H):
                        acc = acc + rows_v[j * H + k, sl]
                    h_v[c * CHUNK + j, sl] = acc * scale
            return carry

        lax.fori_loop(0, n_chunks, body, 0)
        pltpu.sync_copy(h_v, out_hbm.at[pl.ds(wid * b_per_w, b_per_w)])

    return emb_kernel(x_r, table)


def _mlp_softmax(h, W1, b1, W2, b2):
    B, D = h.shape
    BLK = 2048

    def body(h_ref, w1_ref, b1_ref, w2_ref, b2_ref, o_ref):
        z = jnp.dot(h_ref[...], w1_ref[...], preferred_element_type=jnp.float32)
        z = z + b1_ref[...]
        z = jnp.dot(z, w2_ref[...], preferred_element_type=jnp.float32)
        z = z + b2_ref[...]
        z = z - jnp.max(z, axis=1, keepdims=True)
        e = jnp.exp(z)
        o_ref[...] = e / jnp.sum(e, axis=1, keepdims=True)

    return pl.pallas_call(
        body,
        grid=(B // BLK,),
        in_specs=[
            pl.BlockSpec((BLK, D), lambda i: (i, 0)),
            pl.BlockSpec((D, D), lambda i: (0, 0)),
            pl.BlockSpec((1, D), lambda i: (0, 0)),
            pl.BlockSpec((D, D), lambda i: (0, 0)),
            pl.BlockSpec((1, D), lambda i: (0, 0)),
        ],
        out_specs=pl.BlockSpec((BLK, D), lambda i: (i, 0)),
        out_shape=jax.ShapeDtypeStruct((B, D), jnp.float32),
    )(h, W1.T, b1.reshape(1, D), W2.T, b2.reshape(1, D))


def kernel(x, table, W1, b1, W2, b2):
    h = _embedding_bag_mean(x, table)
    return _mlp_softmax(h, W1, b1, W2, b2)


# speedup vs baseline: 2.3437x; 1.1500x over previous
import functools

import jax
import jax.numpy as jnp
from jax import lax
from jax.experimental import pallas as pl
from jax.experimental.pallas import tpu as pltpu
from jax.experimental.pallas import tpu_sc as plsc


def _repack_table(table):
    V, D = table.shape
    BLKV = 2048
    grid = (V + BLKV - 1) // BLKV

    def body(tT_ref, out_ref):
        out_ref[:, 0:D] = tT_ref[...].T

    return pl.pallas_call(
        body,
        grid=(grid,),
        in_specs=[pl.BlockSpec((D, BLKV), lambda i: (0, i))],
        out_specs=pl.BlockSpec((BLKV, 128), lambda i: (i, 0)),
        out_shape=jax.ShapeDtypeStruct((V, 128), jnp.float32),
    )(table.T)


def _embedding_bag_mean(x, table_r, D):
    B, H = x.shape
    W = table_r.shape[1]
    info = plsc.get_sparse_core_info()
    NC, NS, L = info.num_cores, info.num_subcores, info.num_lanes
    NW = NC * NS
    b_per_w = B // NW
    CHUNK = 2
    IPC = CHUNK * H
    n_chunks = b_per_w // CHUNK

    x_r = x.reshape(NW, n_chunks, IPC).astype(jnp.int32)
    mesh = plsc.VectorSubcoreMesh(core_axis_name="c", subcore_axis_name="s")

    @functools.partial(
        pl.kernel,
        mesh=mesh,
        out_type=jax.ShapeDtypeStruct((B, D), jnp.float32),
        scratch_types=[
            pltpu.VMEM((n_chunks, IPC), jnp.int32),
            pltpu.VMEM((2, IPC, W), jnp.float32),
            pltpu.VMEM((b_per_w, D), jnp.float32),
            pltpu.SemaphoreType.DMA,
            pltpu.SemaphoreType.DMA,
        ],
        compiler_params=pltpu.CompilerParams(use_tc_tiling_on_sc=False),
    )
    def emb_kernel(x_hbm, table_hbm, out_hbm, idx_v, rows_v, h_v, sem0, sem1):
        wid = lax.axis_index("s") * NC + lax.axis_index("c")
        pltpu.sync_copy(x_hbm.at[wid], idx_v)
        scale = jnp.float32(1.0 / H)
        sems = (sem0, sem1)

        def accum(c, slot):
            for j in range(CHUNK):
                for dd in range(D // L):
                    sl = pl.ds(dd * L, L)
                    acc = rows_v[slot, j * H, sl]
                    for k in range(1, H):
                        acc = acc + rows_v[slot, j * H + k, sl]
                    h_v[c * CHUNK + j, sl] = acc * scale

        def fire(c, slot):
            pltpu.async_copy(table_hbm.at[idx_v.at[c]], rows_v.at[slot], sems[slot])

        fire(0, 0)

        def body(i, carry):
            c = 2 * i
            fire(c + 1, 1)
            pltpu.make_async_copy(table_hbm.at[idx_v.at[c]], rows_v.at[0], sems[0]).wait()
            accum(c, 0)

            @pl.when(i < n_chunks // 2 - 1)
            def _():
                fire(c + 2, 0)

            pltpu.make_async_copy(table_hbm.at[idx_v.at[c + 1]], rows_v.at[1], sems[1]).wait()
            accum(c + 1, 1)
            return carry

        lax.fori_loop(0, n_chunks // 2, body, 0)
        pltpu.sync_copy(h_v, out_hbm.at[pl.ds(wid * b_per_w, b_per_w)])

    return emb_kernel(x_r, table_r)


def _mlp_softmax(h, W1, b1, W2, b2):
    B, D = h.shape
    BLK = 2048

    def body(h_ref, w1_ref, b1_ref, w2_ref, b2_ref, o_ref):
        z = jnp.dot(h_ref[...], w1_ref[...], preferred_element_type=jnp.float32)
        z = z + b1_ref[...]
        z = jnp.dot(z, w2_ref[...], preferred_element_type=jnp.float32)
        z = z + b2_ref[...]
        z = z - jnp.max(z, axis=1, keepdims=True)
        e = jnp.exp(z)
        o_ref[...] = e / jnp.sum(e, axis=1, keepdims=True)

    return pl.pallas_call(
        body,
        grid=(B // BLK,),
        in_specs=[
            pl.BlockSpec((BLK, D), lambda i: (i, 0)),
            pl.BlockSpec((D, D), lambda i: (0, 0)),
            pl.BlockSpec((1, D), lambda i: (0, 0)),
            pl.BlockSpec((D, D), lambda i: (0, 0)),
            pl.BlockSpec((1, D), lambda i: (0, 0)),
        ],
        out_specs=pl.BlockSpec((BLK, D), lambda i: (i, 0)),
        out_shape=jax.ShapeDtypeStruct((B, D), jnp.float32),
    )(h, W1.T, b1.reshape(1, D), W2.T, b2.reshape(1, D))


def kernel(x, table, W1, b1, W2, b2):
    table_r = _repack_table(table)
    h = _embedding_bag_mean(x, table_r, table.shape[1])
    return _mlp_softmax(h, W1, b1, W2, b2)


# speedup vs baseline: 2.6282x; 1.1214x over previous
import functools

import jax
import jax.numpy as jnp
from jax import lax
from jax.experimental import pallas as pl
from jax.experimental.pallas import tpu as pltpu
from jax.experimental.pallas import tpu_sc as plsc


def _repack_table(table):
    V, D = table.shape
    BLKV = 2048
    grid = (V + BLKV - 1) // BLKV

    def body(tT_ref, eye_ref, out_ref):
        out_ref[:, 0:D] = jax.lax.dot_general(
            tT_ref[...], eye_ref[...],
            dimension_numbers=(((0,), (0,)), ((), ())),
            preferred_element_type=jnp.float32,
        )

    return pl.pallas_call(
        body,
        grid=(grid,),
        in_specs=[
            pl.BlockSpec((D, BLKV), lambda i: (0, i)),
            pl.BlockSpec((D, D), lambda i: (0, 0)),
        ],
        out_specs=pl.BlockSpec((BLKV, 128), lambda i: (i, 0)),
        out_shape=jax.ShapeDtypeStruct((V, 128), jnp.float32),
        compiler_params=pltpu.CompilerParams(fuse_transposed_lhs_in_matmul=True),
    )(table.T, jnp.eye(D, dtype=jnp.float32))


def _embedding_bag_mean(x, table_r, D):
    B, H = x.shape
    W = table_r.shape[1]
    info = plsc.get_sparse_core_info()
    NC, NS, L = info.num_cores, info.num_subcores, info.num_lanes
    NW = NC * NS
    b_per_w = B // NW
    CHUNK = 2
    IPC = CHUNK * H
    n_chunks = b_per_w // CHUNK

    x_r = x.reshape(NW, n_chunks, IPC).astype(jnp.int32)
    mesh = plsc.VectorSubcoreMesh(core_axis_name="c", subcore_axis_name="s")

    @functools.partial(
        pl.kernel,
        mesh=mesh,
        out_type=jax.ShapeDtypeStruct((B, D), jnp.float32),
        scratch_types=[
            pltpu.VMEM((n_chunks, IPC), jnp.int32),
            pltpu.VMEM((2, IPC, W), jnp.float32),
            pltpu.VMEM((b_per_w, D), jnp.float32),
            pltpu.SemaphoreType.DMA,
            pltpu.SemaphoreType.DMA,
        ],
        compiler_params=pltpu.CompilerParams(use_tc_tiling_on_sc=False),
    )
    def emb_kernel(x_hbm, table_hbm, out_hbm, idx_v, rows_v, h_v, sem0, sem1):
        wid = lax.axis_index("s") * NC + lax.axis_index("c")
        pltpu.sync_copy(x_hbm.at[wid], idx_v)
        scale = jnp.float32(1.0 / H)
        sems = (sem0, sem1)

        def accum(c, slot):
            for j in range(CHUNK):
                for dd in range(D // L):
                    sl = pl.ds(dd * L, L)
                    vals = [rows_v[slot, j * H + k, sl] for k in range(H)]
                    while len(vals) > 1:
                        nxt = [vals[i] + vals[i + 1]
                               for i in range(0, len(vals) - 1, 2)]
                        if len(vals) % 2:
                            nxt.append(vals[-1])
                        vals = nxt
                    h_v[c * CHUNK + j, sl] = vals[0] * scale

        def fire(c, slot):
            pltpu.async_copy(table_hbm.at[idx_v.at[c]], rows_v.at[slot], sems[slot])

        fire(0, 0)

        def body(i, carry):
            c = 2 * i
            fire(c + 1, 1)
            pltpu.make_async_copy(table_hbm.at[idx_v.at[c]], rows_v.at[0], sems[0]).wait()
            accum(c, 0)

            @pl.when(i < n_chunks // 2 - 1)
            def _():
                fire(c + 2, 0)

            pltpu.make_async_copy(table_hbm.at[idx_v.at[c + 1]], rows_v.at[1], sems[1]).wait()
            accum(c + 1, 1)
            return carry

        lax.fori_loop(0, n_chunks // 2, body, 0)
        pltpu.sync_copy(h_v, out_hbm.at[pl.ds(wid * b_per_w, b_per_w)])

    return emb_kernel(x_r, table_r)


def _mlp_softmax(h, W1, b1, W2, b2):
    B, D = h.shape
    BLK = 2048

    def body(h_ref, w1_ref, b1_ref, w2_ref, b2_ref, o_ref):
        z = jnp.dot(h_ref[...], w1_ref[...], preferred_element_type=jnp.float32)
        z = z + b1_ref[...]
        z = jnp.dot(z, w2_ref[...], preferred_element_type=jnp.float32)
        z = z + b2_ref[...]
        z = z - jnp.max(z, axis=1, keepdims=True)
        e = jnp.exp(z)
        o_ref[...] = e / jnp.sum(e, axis=1, keepdims=True)

    return pl.pallas_call(
        body,
        grid=(B // BLK,),
        in_specs=[
            pl.BlockSpec((BLK, D), lambda i: (i, 0)),
            pl.BlockSpec((D, D), lambda i: (0, 0)),
            pl.BlockSpec((1, D), lambda i: (0, 0)),
            pl.BlockSpec((D, D), lambda i: (0, 0)),
            pl.BlockSpec((1, D), lambda i: (0, 0)),
        ],
        out_specs=pl.BlockSpec((BLK, D), lambda i: (i, 0)),
        out_shape=jax.ShapeDtypeStruct((B, D), jnp.float32),
    )(h, W1.T, b1.reshape(1, D), W2.T, b2.reshape(1, D))


def kernel(x, table, W1, b1, W2, b2):
    table_r = _repack_table(table)
    h = _embedding_bag_mean(x, table_r, table.shape[1])
    return _mlp_softmax(h, W1, b1, W2, b2)


# speedup vs baseline: 3.6964x; 1.4065x over previous
import functools

import jax
import jax.numpy as jnp
from jax import lax
from jax.experimental import pallas as pl
from jax.experimental.pallas import tpu as pltpu
from jax.experimental.pallas import tpu_sc as plsc


def _repack_table(table):
    V, D = table.shape
    BLKV = 4096
    grid = (V + BLKV - 1) // BLKV
    P = BLKV // 2

    def body(tT_ref, eye_ref, out_ref):
        t = jax.lax.dot_general(
            tT_ref[...], eye_ref[...],
            dimension_numbers=(((0,), (0,)), ((), ())),
            preferred_element_type=jnp.float32,
        )
        out_ref[:, 0:D] = t[0:P, :]
        out_ref[:, D:2 * D] = t[P:BLKV, :]

    packed = pl.pallas_call(
        body,
        grid=(grid,),
        in_specs=[
            pl.BlockSpec((D, BLKV), lambda i: (0, i)),
            pl.BlockSpec((D, D), lambda i: (0, 0)),
        ],
        out_specs=pl.BlockSpec((P, 2 * D), lambda i: (i, 0)),
        out_shape=jax.ShapeDtypeStruct((grid * P, 2 * D), jnp.float32),
        compiler_params=pltpu.CompilerParams(fuse_transposed_lhs_in_matmul=True),
    )(table.T, jnp.eye(D, dtype=jnp.float32))
    return packed.reshape(grid * BLKV, D)


def _stage_index(x, BLKV=4096):
    P = BLKV // 2
    return (x & ~(BLKV - 1)) + ((x & (P - 1)) << 1) + ((x >> 11) & 1)


def _embedding_bag_mean(x, table_r, D):
    B, H = x.shape
    W = table_r.shape[1]
    info = plsc.get_sparse_core_info()
    NC, NS, L = info.num_cores, info.num_subcores, info.num_lanes
    NW = NC * NS
    b_per_w = B // NW
    CHUNK = 2
    IPC = CHUNK * H
    n_chunks = b_per_w // CHUNK

    x_r = x.reshape(NW, n_chunks, IPC).astype(jnp.int32)
    mesh = plsc.VectorSubcoreMesh(core_axis_name="c", subcore_axis_name="s")

    @functools.partial(
        pl.kernel,
        mesh=mesh,
        out_type=jax.ShapeDtypeStruct((B, D), jnp.float32),
        scratch_types=[
            pltpu.VMEM((n_chunks, IPC), jnp.int32),
            pltpu.VMEM((2, IPC, W), jnp.float32),
            pltpu.VMEM((b_per_w, D), jnp.float32),
            pltpu.SemaphoreType.DMA,
            pltpu.SemaphoreType.DMA,
        ],
        compiler_params=pltpu.CompilerParams(use_tc_tiling_on_sc=False),
    )
    def emb_kernel(x_hbm, table_hbm, out_hbm, idx_v, rows_v, h_v, sem0, sem1):
        wid = lax.axis_index("s") * NC + lax.axis_index("c")
        pltpu.sync_copy(x_hbm.at[wid], idx_v)
        scale = jnp.float32(1.0 / H)
        sems = (sem0, sem1)

        def accum(c, slot):
            for j in range(CHUNK):
                for dd in range(D // L):
                    sl = pl.ds(dd * L, L)
                    vals = [rows_v[slot, j * H + k, sl] for k in range(H)]
                    while len(vals) > 1:
                        nxt = [vals[i] + vals[i + 1]
                               for i in range(0, len(vals) - 1, 2)]
                        if len(vals) % 2:
                            nxt.append(vals[-1])
                        vals = nxt
                    h_v[c * CHUNK + j, sl] = vals[0] * scale

        def fire(c, slot):
            pltpu.async_copy(table_hbm.at[idx_v.at[c]], rows_v.at[slot], sems[slot])

        fire(0, 0)

        def body(i, carry):
            c = 2 * i
            fire(c + 1, 1)
            pltpu.make_async_copy(table_hbm.at[idx_v.at[c]], rows_v.at[0], sems[0]).wait()
            accum(c, 0)

            @pl.when(i < n_chunks // 2 - 1)
            def _():
                fire(c + 2, 0)

            pltpu.make_async_copy(table_hbm.at[idx_v.at[c + 1]], rows_v.at[1], sems[1]).wait()
            accum(c + 1, 1)
            return carry

        lax.fori_loop(0, n_chunks // 2, body, 0)
        pltpu.sync_copy(h_v, out_hbm.at[pl.ds(wid * b_per_w, b_per_w)])

    return emb_kernel(x_r, table_r)


def _mlp_softmax(h, W1, b1, W2, b2):
    B, D = h.shape
    BLK = 2048

    def body(h_ref, w1_ref, b1_ref, w2_ref, b2_ref, o_ref):
        z = jnp.dot(h_ref[...], w1_ref[...], preferred_element_type=jnp.float32)
        z = z + b1_ref[...]
        z = jnp.dot(z, w2_ref[...], preferred_element_type=jnp.float32)
        z = z + b2_ref[...]
        z = z - jnp.max(z, axis=1, keepdims=True)
        e = jnp.exp(z)
        o_ref[...] = e / jnp.sum(e, axis=1, keepdims=True)

    return pl.pallas_call(
        body,
        grid=(B // BLK,),
        in_specs=[
            pl.BlockSpec((BLK, D), lambda i: (i, 0)),
            pl.BlockSpec((D, D), lambda i: (0, 0)),
            pl.BlockSpec((1, D), lambda i: (0, 0)),
            pl.BlockSpec((D, D), lambda i: (0, 0)),
            pl.BlockSpec((1, D), lambda i: (0, 0)),
        ],
        out_specs=pl.BlockSpec((BLK, D), lambda i: (i, 0)),
        out_shape=jax.ShapeDtypeStruct((B, D), jnp.float32),
    )(h, W1.T, b1.reshape(1, D), W2.T, b2.reshape(1, D))


def kernel(x, table, W1, b1, W2, b2):
    table_r = _repack_table(table)
    h = _embedding_bag_mean(_stage_index(x), table_r, table.shape[1])
    return _mlp_softmax(h, W1, b1, W2, b2)


# speedup vs baseline: 4.0380x; 1.0924x over previous
import functools

import jax
import jax.numpy as jnp
import numpy as np
from jax import lax
from jax.experimental import pallas as pl
from jax.experimental.pallas import tpu as pltpu
from jax.experimental.pallas import tpu_sc as plsc


def _repack_table(table):
    V, D = table.shape
    BLKV = 8192
    grid = (V + BLKV - 1) // BLKV
    P = BLKV // 4

    def body(tT_ref, eye_ref, out_ref):
        t = jax.lax.dot_general(
            tT_ref[...], eye_ref[...],
            dimension_numbers=(((0,), (0,)), ((), ())),
            preferred_element_type=jnp.float32,
        )
        lo = jax.lax.bitcast_convert_type(
            t[:, 0:D // 2].astype(jnp.bfloat16), jnp.uint16).astype(jnp.uint32)
        hi = jax.lax.bitcast_convert_type(
            t[:, D // 2:D].astype(jnp.bfloat16), jnp.uint16).astype(jnp.uint32)
        tp = jax.lax.bitcast_convert_type(lo | (hi << 16), jnp.float32)
        for q in range(4):
            out_ref[:, q * (D // 2):(q + 1) * (D // 2)] = (
                tp[q * P:(q + 1) * P, :])

    packed = pl.pallas_call(
        body,
        grid=(grid,),
        in_specs=[
            pl.BlockSpec((D, BLKV), lambda i: (0, i)),
            pl.BlockSpec((D, D), lambda i: (0, 0)),
        ],
        out_specs=pl.BlockSpec((P, 2 * D), lambda i: (i, 0)),
        out_shape=jax.ShapeDtypeStruct((grid * P, 2 * D), jnp.float32),
        compiler_params=pltpu.CompilerParams(fuse_transposed_lhs_in_matmul=True),
    )(table.T, jnp.eye(D, dtype=jnp.float32))
    return packed.reshape(grid * BLKV, D // 2)


def _stage_index(x, BLKV=8192):
    P = BLKV // 4
    return (x & ~(BLKV - 1)) + ((x & (P - 1)) << 2) + ((x >> 11) & 3)


def _embedding_bag_mean(x, table_r, D):
    B, H = x.shape
    W = table_r.shape[1]
    info = plsc.get_sparse_core_info()
    NC, NS, L = info.num_cores, info.num_subcores, info.num_lanes
    NW = NC * NS
    b_per_w = B // NW
    CHUNK = 2
    IPC = CHUNK * H
    n_chunks = b_per_w // CHUNK

    x_r = x.reshape(NW, n_chunks, IPC).astype(jnp.int32)
    mesh = plsc.VectorSubcoreMesh(core_axis_name="c", subcore_axis_name="s")

    @functools.partial(
        pl.kernel,
        mesh=mesh,
        out_type=jax.ShapeDtypeStruct((B, D), jnp.float32),
        scratch_types=[
            pltpu.VMEM((n_chunks, IPC), jnp.int32),
            pltpu.VMEM((2, IPC, W), jnp.float32),
            pltpu.VMEM((b_per_w, D), jnp.float32),
            pltpu.SemaphoreType.DMA,
            pltpu.SemaphoreType.DMA,
        ],
        compiler_params=pltpu.CompilerParams(
            use_tc_tiling_on_sc=False, needs_layout_passes=False),
    )
    def emb_kernel(x_hbm, table_hbm, out_hbm, idx_v, rows_v, h_v, sem0, sem1):
        wid = lax.axis_index("s") * NC + lax.axis_index("c")
        pltpu.sync_copy(x_hbm.at[wid], idx_v)
        scale = jnp.float32(1.0 / H)
        sems = (sem0, sem1)

        def tree(vals):
            while len(vals) > 1:
                nxt = [vals[i] + vals[i + 1]
                       for i in range(0, len(vals) - 1, 2)]
                if len(vals) % 2:
                    nxt.append(vals[-1])
                vals = nxt
            return vals[0]

        def accum(c, slot):
            for j in range(CHUNK):
                streams = [[], [], [], []]
                for k in range(H):
                    for half in range(W // L):
                        s = rows_v[slot, j * H + k, pl.ds(half * L, L)]
                        bb = plsc.bitcast(s, jnp.bfloat16)
                        a, b = plsc.unpack(
                            bb, format=plsc.PackFormat.INTERLEAVED)
                        streams[half * 2].append(a)
                        streams[half * 2 + 1].append(b)
                for q in range(4):
                    h_v[c * CHUNK + j, pl.ds(q * L, L)] = tree(streams[q]) * scale

        def fire(c, slot):
            pltpu.async_copy(table_hbm.at[idx_v.at[c]], rows_v.at[slot], sems[slot])

        fire(0, 0)

        def body(i, carry):
            c = 2 * i
            fire(c + 1, 1)
            pltpu.make_async_copy(table_hbm.at[idx_v.at[c]], rows_v.at[0], sems[0]).wait()
            accum(c, 0)

            @pl.when(i < n_chunks // 2 - 1)
            def _():
                fire(c + 2, 0)

            pltpu.make_async_copy(table_hbm.at[idx_v.at[c + 1]], rows_v.at[1], sems[1]).wait()
            accum(c + 1, 1)
            return carry

        lax.fori_loop(0, n_chunks // 2, body, 0)
        pltpu.sync_copy(h_v, out_hbm.at[pl.ds(wid * b_per_w, b_per_w)])

    return emb_kernel(x_r, table_r)


def _mlp_softmax(h, w1t, b1, w2t, b2):
    B, D = h.shape
    BLK = 2048

    def body(h_ref, w1_ref, b1_ref, w2_ref, b2_ref, o_ref):
        z = jnp.dot(h_ref[...], w1_ref[...], preferred_element_type=jnp.float32)
        z = z + b1_ref[...]
        z = jnp.dot(z, w2_ref[...], preferred_element_type=jnp.float32)
        z = z + b2_ref[...]
        z = z - jnp.max(z, axis=1, keepdims=True)
        e = jnp.exp(z)
        o_ref[...] = e / jnp.sum(e, axis=1, keepdims=True)

    return pl.pallas_call(
        body,
        grid=(B // BLK,),
        in_specs=[
            pl.BlockSpec((BLK, D), lambda i: (i, 0)),
            pl.BlockSpec((D, D), lambda i: (0, 0)),
            pl.BlockSpec((1, D), lambda i: (0, 0)),
            pl.BlockSpec((D, D), lambda i: (0, 0)),
            pl.BlockSpec((1, D), lambda i: (0, 0)),
        ],
        out_specs=pl.BlockSpec((BLK, D), lambda i: (i, 0)),
        out_shape=jax.ShapeDtypeStruct((B, D), jnp.float32),
    )(h, w1t, b1.reshape(1, D), w2t, b2.reshape(1, D))


_H_PERM = np.concatenate([
    np.arange(0, 16), np.arange(32, 48),
    np.arange(16, 32), np.arange(48, 64),
])


def kernel(x, table, W1, b1, W2, b2):
    table_r = _repack_table(table)
    h = _embedding_bag_mean(_stage_index(x), table_r, table.shape[1])
    return _mlp_softmax(h, W1.T[_H_PERM, :], b1, W2.T, b2)


# speedup vs baseline: 4.0670x; 1.0072x over previous
import functools

import jax
import jax.numpy as jnp
import numpy as np
from jax import lax
from jax.experimental import pallas as pl
from jax.experimental.pallas import tpu as pltpu
from jax.experimental.pallas import tpu_sc as plsc


def _repack_table(table):
    V, D = table.shape
    BLKV = 8192
    grid = (V + BLKV - 1) // BLKV
    P = BLKV // 4

    def body(tT_ref, eye_ref, out_ref):
        t = jax.lax.dot_general(
            tT_ref[...], eye_ref[...],
            dimension_numbers=(((0,), (0,)), ((), ())),
            preferred_element_type=jnp.float32,
        )
        tu = jax.lax.bitcast_convert_type(t, jnp.uint32)
        lo = tu[:, 0:D // 2] >> 16
        hi = tu[:, D // 2:D] & jnp.uint32(0xFFFF0000)
        tp = jax.lax.bitcast_convert_type(lo | hi, jnp.float32)
        for q in range(4):
            out_ref[:, q * (D // 2):(q + 1) * (D // 2)] = (
                tp[q * P:(q + 1) * P, :])

    packed = pl.pallas_call(
        body,
        grid=(grid,),
        in_specs=[
            pl.BlockSpec((D, BLKV), lambda i: (0, i)),
            pl.BlockSpec((D, D), lambda i: (0, 0)),
        ],
        out_specs=pl.BlockSpec((P, 2 * D), lambda i: (i, 0)),
        out_shape=jax.ShapeDtypeStruct((grid * P, 2 * D), jnp.float32),
        compiler_params=pltpu.CompilerParams(fuse_transposed_lhs_in_matmul=True),
    )(table.T, jnp.eye(D, dtype=jnp.float32))
    return packed.reshape(grid * BLKV, D // 2)


def _stage_index(x, BLKV=8192):
    P = BLKV // 4
    return (x & ~(BLKV - 1)) + ((x & (P - 1)) << 2) + ((x >> 11) & 3)


def _embedding_bag_mean(x, table_r, D):
    B, H = x.shape
    W = table_r.shape[1]
    info = plsc.get_sparse_core_info()
    NC, NS, L = info.num_cores, info.num_subcores, info.num_lanes
    NW = NC * NS
    b_per_w = B // NW
    CHUNK = 2
    IPC = CHUNK * H
    n_chunks = b_per_w // CHUNK

    x_r = x.reshape(NW, n_chunks, IPC).astype(jnp.int32)
    mesh = plsc.VectorSubcoreMesh(core_axis_name="c", subcore_axis_name="s")

    @functools.partial(
        pl.kernel,
        mesh=mesh,
        out_type=jax.ShapeDtypeStruct((B, D), jnp.float32),
        scratch_types=[
            pltpu.VMEM((n_chunks, IPC), jnp.int32),
            pltpu.VMEM((2, IPC, W), jnp.float32),
            pltpu.VMEM((b_per_w, D), jnp.float32),
            pltpu.SemaphoreType.DMA,
            pltpu.SemaphoreType.DMA,
        ],
        compiler_params=pltpu.CompilerParams(
            use_tc_tiling_on_sc=False, needs_layout_passes=False),
    )
    def emb_kernel(x_hbm, table_hbm, out_hbm, idx_v, rows_v, h_v, sem0, sem1):
        wid = lax.axis_index("s") * NC + lax.axis_index("c")
        pltpu.sync_copy(x_hbm.at[wid], idx_v)
        scale = jnp.float32(1.0 / H)
        sems = (sem0, sem1)

        def tree(vals):
            while len(vals) > 1:
                nxt = [vals[i] + vals[i + 1]
                       for i in range(0, len(vals) - 1, 2)]
                if len(vals) % 2:
                    nxt.append(vals[-1])
                vals = nxt
            return vals[0]

        def accum(c, slot):
            for j in range(CHUNK):
                for half in range(W // L):
                    vals = [
                        plsc.bitcast(
                            rows_v[slot, j * H + k, pl.ds(half * L, L)],
                            jnp.bfloat16)
                        for k in range(H)
                    ]
                    a, b = plsc.unpack(
                        tree(vals), format=plsc.PackFormat.INTERLEAVED)
                    h_v[c * CHUNK + j, pl.ds(2 * half * L, L)] = a * scale
                    h_v[c * CHUNK + j, pl.ds((2 * half + 1) * L, L)] = b * scale

        def fire(c, slot):
            pltpu.async_copy(table_hbm.at[idx_v.at[c]], rows_v.at[slot], sems[slot])

        fire(0, 0)

        def body(i, carry):
            c = 2 * i
            fire(c + 1, 1)
            pltpu.make_async_copy(table_hbm.at[idx_v.at[c]], rows_v.at[0], sems[0]).wait()
            accum(c, 0)

            @pl.when(i < n_chunks // 2 - 1)
            def _():
                fire(c + 2, 0)

            pltpu.make_async_copy(table_hbm.at[idx_v.at[c + 1]], rows_v.at[1], sems[1]).wait()
            accum(c + 1, 1)
            return carry

        lax.fori_loop(0, n_chunks // 2, body, 0)
        pltpu.sync_copy(h_v, out_hbm.at[pl.ds(wid * b_per_w, b_per_w)])

    return emb_kernel(x_r, table_r)


def _mlp_softmax(h, w1t, b1, w2t, b2):
    B, D = h.shape
    BLK = 2048

    def body(h_ref, w1_ref, b1_ref, w2_ref, b2_ref, o_ref):
        z = jnp.dot(h_ref[...], w1_ref[...], preferred_element_type=jnp.float32)
        z = z + b1_ref[...]
        z = jnp.dot(z, w2_ref[...], preferred_element_type=jnp.float32)
        z = z + b2_ref[...]
        z = z - jnp.max(z, axis=1, keepdims=True)
        e = jnp.exp(z)
        o_ref[...] = e / jnp.sum(e, axis=1, keepdims=True)

    return pl.pallas_call(
        body,
        grid=(B // BLK,),
        in_specs=[
            pl.BlockSpec((BLK, D), lambda i: (i, 0)),
            pl.BlockSpec((D, D), lambda i: (0, 0)),
            pl.BlockSpec((1, D), lambda i: (0, 0)),
            pl.BlockSpec((D, D), lambda i: (0, 0)),
            pl.BlockSpec((1, D), lambda i: (0, 0)),
        ],
        out_specs=pl.BlockSpec((BLK, D), lambda i: (i, 0)),
        out_shape=jax.ShapeDtypeStruct((B, D), jnp.float32),
    )(h, w1t, b1.reshape(1, D), w2t, b2.reshape(1, D))


_H_PERM = np.concatenate([
    np.arange(0, 16), np.arange(32, 48),
    np.arange(16, 32), np.arange(48, 64),
])


def kernel(x, table, W1, b1, W2, b2):
    table_r = _repack_table(table)
    h = _embedding_bag_mean(_stage_index(x), table_r, table.shape[1])
    return _mlp_softmax(h, W1.T[_H_PERM, :], b1, W2.T, b2)


# speedup vs baseline: 4.0949x; 1.0068x over previous
import functools

import jax
import jax.numpy as jnp
import numpy as np
from jax import lax
from jax.experimental import pallas as pl
from jax.experimental.pallas import tpu as pltpu
from jax.experimental.pallas import tpu_sc as plsc


def _repack_table(table):
    V, D = table.shape
    BLKV = 16384
    grid = (V + BLKV - 1) // BLKV
    P = BLKV // 4

    def body(tT_ref, eye_ref, out_ref):
        t = jax.lax.dot_general(
            tT_ref[...], eye_ref[...],
            dimension_numbers=(((0,), (0,)), ((), ())),
            preferred_element_type=jnp.float32,
        )
        tu = jax.lax.bitcast_convert_type(t, jnp.uint32)
        lo = tu[:, 0:D // 2] >> 16
        hi = tu[:, D // 2:D] & jnp.uint32(0xFFFF0000)
        tp = jax.lax.bitcast_convert_type(lo | hi, jnp.float32)
        for q in range(4):
            out_ref[:, q * (D // 2):(q + 1) * (D // 2)] = (
                tp[q * P:(q + 1) * P, :])

    packed = pl.pallas_call(
        body,
        grid=(grid,),
        in_specs=[
            pl.BlockSpec((D, BLKV), lambda i: (0, i)),
            pl.BlockSpec((D, D), lambda i: (0, 0)),
        ],
        out_specs=pl.BlockSpec((P, 2 * D), lambda i: (i, 0)),
        out_shape=jax.ShapeDtypeStruct((grid * P, 2 * D), jnp.float32),
        compiler_params=pltpu.CompilerParams(fuse_transposed_lhs_in_matmul=True),
    )(table.T, jnp.eye(D, dtype=jnp.float32))
    return packed.reshape(grid * BLKV, D // 2)


def _stage_index(x, BLKV=16384):
    P = BLKV // 4
    sh = P.bit_length() - 1
    return (x & ~(BLKV - 1)) + ((x & (P - 1)) << 2) + ((x >> sh) & 3)


def _embedding_bag_mean(x, table_r, D):
    B, H = x.shape
    W = table_r.shape[1]
    info = plsc.get_sparse_core_info()
    NC, NS, L = info.num_cores, info.num_subcores, info.num_lanes
    NW = NC * NS
    b_per_w = B // NW
    CHUNK = 2
    IPC = CHUNK * H
    n_chunks = b_per_w // CHUNK

    x_r = x.reshape(NW, n_chunks, IPC).astype(jnp.int32)
    mesh = plsc.VectorSubcoreMesh(core_axis_name="c", subcore_axis_name="s")

    @functools.partial(
        pl.kernel,
        mesh=mesh,
        out_type=jax.ShapeDtypeStruct((B, D), jnp.float32),
        scratch_types=[
            pltpu.VMEM((n_chunks, IPC), jnp.int32),
            pltpu.VMEM((2, IPC, W), jnp.float32),
            pltpu.VMEM((b_per_w, D), jnp.float32),
            pltpu.SemaphoreType.DMA,
            pltpu.SemaphoreType.DMA,
        ],
        compiler_params=pltpu.CompilerParams(
            use_tc_tiling_on_sc=False, needs_layout_passes=False),
    )
    def emb_kernel(x_hbm, table_hbm, out_hbm, idx_v, rows_v, h_v, sem0, sem1):
        wid = lax.axis_index("s") * NC + lax.axis_index("c")
        pltpu.sync_copy(x_hbm.at[wid], idx_v)
        scale = jnp.float32(1.0 / H)
        sems = (sem0, sem1)

        def tree(vals):
            while len(vals) > 1:
                nxt = [vals[i] + vals[i + 1]
                       for i in range(0, len(vals) - 1, 2)]
                if len(vals) % 2:
                    nxt.append(vals[-1])
                vals = nxt
            return vals[0]

        def accum(c, slot):
            for j in range(CHUNK):
                for half in range(W // L):
                    vals = [
                        plsc.bitcast(
                            rows_v[slot, j * H + k, pl.ds(half * L, L)],
                            jnp.bfloat16)
                        for k in range(H)
                    ]
                    a, b = plsc.unpack(
                        tree(vals), format=plsc.PackFormat.INTERLEAVED)
                    h_v[c * CHUNK + j, pl.ds(2 * half * L, L)] = a * scale
                    h_v[c * CHUNK + j, pl.ds((2 * half + 1) * L, L)] = b * scale

        def fire(c, slot):
            pltpu.async_copy(table_hbm.at[idx_v.at[c]], rows_v.at[slot], sems[slot])

        fire(0, 0)

        def body(i, carry):
            c = 2 * i
            fire(c + 1, 1)
            pltpu.make_async_copy(table_hbm.at[idx_v.at[c]], rows_v.at[0], sems[0]).wait()
            accum(c, 0)

            @pl.when(i < n_chunks // 2 - 1)
            def _():
                fire(c + 2, 0)

            pltpu.make_async_copy(table_hbm.at[idx_v.at[c + 1]], rows_v.at[1], sems[1]).wait()
            accum(c + 1, 1)
            return carry

        lax.fori_loop(0, n_chunks // 2, body, 0)
        pltpu.sync_copy(h_v, out_hbm.at[pl.ds(wid * b_per_w, b_per_w)])

    return emb_kernel(x_r, table_r)


def _mlp_softmax(h, w1t, b1, w2t, b2):
    B, D = h.shape
    BLK = 2048

    def body(h_ref, w1_ref, b1_ref, w2_ref, b2_ref, o_ref):
        z = jnp.dot(h_ref[...], w1_ref[...], preferred_element_type=jnp.float32)
        z = z + b1_ref[...]
        z = jnp.dot(z, w2_ref[...], preferred_element_type=jnp.float32)
        z = z + b2_ref[...]
        z = z - jnp.max(z, axis=1, keepdims=True)
        e = jnp.exp(z)
        o_ref[...] = e / jnp.sum(e, axis=1, keepdims=True)

    return pl.pallas_call(
        body,
        grid=(B // BLK,),
        in_specs=[
            pl.BlockSpec((BLK, D), lambda i: (i, 0)),
            pl.BlockSpec((D, D), lambda i: (0, 0)),
            pl.BlockSpec((1, D), lambda i: (0, 0)),
            pl.BlockSpec((D, D), lambda i: (0, 0)),
            pl.BlockSpec((1, D), lambda i: (0, 0)),
        ],
        out_specs=pl.BlockSpec((BLK, D), lambda i: (i, 0)),
        out_shape=jax.ShapeDtypeStruct((B, D), jnp.float32),
    )(h, w1t, b1.reshape(1, D), w2t, b2.reshape(1, D))


_H_PERM = np.concatenate([
    np.arange(0, 16), np.arange(32, 48),
    np.arange(16, 32), np.arange(48, 64),
])


def kernel(x, table, W1, b1, W2, b2):
    table_r = _repack_table(table)
    h = _embedding_bag_mean(_stage_index(x), table_r, table.shape[1])
    return _mlp_softmax(h, W1.T[_H_PERM, :], b1, W2.T, b2)


# speedup vs baseline: 4.4127x; 1.0776x over previous
import functools

import jax
import jax.numpy as jnp
import numpy as np
from jax import lax
from jax.experimental import pallas as pl
from jax.experimental.pallas import tpu as pltpu
from jax.experimental.pallas import tpu_sc as plsc


def _repack_table(table):
    V, D = table.shape
    BLKV = 16384
    grid = (V + BLKV - 1) // BLKV
    P = BLKV // 4

    def body(tT_ref, eye_ref, out_ref):
        t = jax.lax.dot_general(
            tT_ref[...], eye_ref[...],
            dimension_numbers=(((0,), (0,)), ((), ())),
            preferred_element_type=jnp.float32,
        )
        tu = jax.lax.bitcast_convert_type(t, jnp.uint32)
        lo = tu[:, 0:D // 2] >> 16
        hi = tu[:, D // 2:D] & jnp.uint32(0xFFFF0000)
        tp = jax.lax.bitcast_convert_type(lo | hi, jnp.float32)
        for q in range(4):
            out_ref[:, q * (D // 2):(q + 1) * (D // 2)] = (
                tp[q * P:(q + 1) * P, :])

    packed = pl.pallas_call(
        body,
        grid=(grid,),
        in_specs=[
            pl.BlockSpec((D, BLKV), lambda i: (0, i)),
            pl.BlockSpec((D, D), lambda i: (0, 0)),
        ],
        out_specs=pl.BlockSpec((P, 2 * D), lambda i: (i, 0)),
        out_shape=jax.ShapeDtypeStruct((grid * P, 2 * D), jnp.float32),
        compiler_params=pltpu.CompilerParams(fuse_transposed_lhs_in_matmul=True),
    )(table.T, jnp.eye(D, dtype=jnp.float32))
    return packed.reshape(grid * BLKV, D // 2)


def _stage_index(x, BLKV=16384):
    P = BLKV // 4
    sh = P.bit_length() - 1
    return (x & ~(BLKV - 1)) + ((x & (P - 1)) << 2) + ((x >> sh) & 3)


def _embedding_bag_mean(x, table_r, D):
    B, H = x.shape
    W = table_r.shape[1]
    info = plsc.get_sparse_core_info()
    NC, NS, L = info.num_cores, info.num_subcores, info.num_lanes
    NW = NC * NS
    b_per_w = B // NW
    CHUNK = 2
    IPC = CHUNK * H
    n_chunks = b_per_w // CHUNK

    x_r = x.reshape(NW, n_chunks, IPC).astype(jnp.int32)
    mesh = plsc.VectorSubcoreMesh(core_axis_name="c", subcore_axis_name="s")

    @functools.partial(
        pl.kernel,
        mesh=mesh,
        out_type=jax.ShapeDtypeStruct((B, D), jnp.float32),
        scratch_types=[
            pltpu.VMEM((n_chunks, IPC), jnp.int32),
            pltpu.VMEM((4, IPC, W), jnp.float32),
            pltpu.VMEM((b_per_w, D), jnp.float32),
            pltpu.SemaphoreType.DMA,
            pltpu.SemaphoreType.DMA,
            pltpu.SemaphoreType.DMA,
            pltpu.SemaphoreType.DMA,
        ],
        compiler_params=pltpu.CompilerParams(
            use_tc_tiling_on_sc=False, needs_layout_passes=False),
    )
    def emb_kernel(x_hbm, table_hbm, out_hbm, idx_v, rows_v, h_v,
                   sem0, sem1, sem2, sem3):
        wid = lax.axis_index("s") * NC + lax.axis_index("c")
        pltpu.sync_copy(x_hbm.at[wid], idx_v)
        scale = jnp.float32(1.0 / H)
        sems = (sem0, sem1, sem2, sem3)

        def tree(vals):
            while len(vals) > 1:
                nxt = [vals[i] + vals[i + 1]
                       for i in range(0, len(vals) - 1, 2)]
                if len(vals) % 2:
                    nxt.append(vals[-1])
                vals = nxt
            return vals[0]

        def accum(c, slot):
            for j in range(CHUNK):
                for half in range(W // L):
                    vals = [
                        plsc.bitcast(
                            rows_v[slot, j * H + k, pl.ds(half * L, L)],
                            jnp.bfloat16)
                        for k in range(H)
                    ]
                    a, b = plsc.unpack(
                        tree(vals), format=plsc.PackFormat.INTERLEAVED)
                    h_v[c * CHUNK + j, pl.ds(2 * half * L, L)] = a * scale
                    h_v[c * CHUNK + j, pl.ds((2 * half + 1) * L, L)] = b * scale

        def fire(c, slot):
            pltpu.async_copy(table_hbm.at[idx_v.at[c]], rows_v.at[slot], sems[slot])

        def wait(c, slot):
            pltpu.make_async_copy(
                table_hbm.at[idx_v.at[c]], rows_v.at[slot], sems[slot]).wait()

        fire(0, 0)
        fire(1, 1)
        fire(2, 2)

        def body(i, carry):
            c = 4 * i
            for s in range(4):
                cc = c + s
                wait(cc, s)
                accum(cc, s)
                nf = cc + 3

                @pl.when(nf < n_chunks)
                def _():
                    fire(nf, (s + 3) % 4)
            return carry

        lax.fori_loop(0, n_chunks // 4, body, 0)
        pltpu.sync_copy(h_v, out_hbm.at[pl.ds(wid * b_per_w, b_per_w)])

    return emb_kernel(x_r, table_r)


def _mlp_softmax(h, w1t, b1, w2t, b2):
    B, D = h.shape
    BLK = 2048

    def body(h_ref, w1_ref, b1_ref, w2_ref, b2_ref, o_ref):
        z = jnp.dot(h_ref[...], w1_ref[...], preferred_element_type=jnp.float32)
        z = z + b1_ref[...]
        z = jnp.dot(z, w2_ref[...], preferred_element_type=jnp.float32)
        z = z + b2_ref[...]
        z = z - jnp.max(z, axis=1, keepdims=True)
        e = jnp.exp(z)
        o_ref[...] = e / jnp.sum(e, axis=1, keepdims=True)

    return pl.pallas_call(
        body,
        grid=(B // BLK,),
        in_specs=[
            pl.BlockSpec((BLK, D), lambda i: (i, 0)),
            pl.BlockSpec((D, D), lambda i: (0, 0)),
            pl.BlockSpec((1, D), lambda i: (0, 0)),
            pl.BlockSpec((D, D), lambda i: (0, 0)),
            pl.BlockSpec((1, D), lambda i: (0, 0)),
        ],
        out_specs=pl.BlockSpec((BLK, D), lambda i: (i, 0)),
        out_shape=jax.ShapeDtypeStruct((B, D), jnp.float32),
    )(h, w1t, b1.reshape(1, D), w2t, b2.reshape(1, D))


_H_PERM = np.concatenate([
    np.arange(0, 16), np.arange(32, 48),
    np.arange(16, 32), np.arange(48, 64),
])


def kernel(x, table, W1, b1, W2, b2):
    table_r = _repack_table(table)
    h = _embedding_bag_mean(_stage_index(x), table_r, table.shape[1])
    return _mlp_softmax(h, W1.T[_H_PERM, :], b1, W2.T, b2)
